# baseline (speedup 1.0000x reference)
import functools

import jax
import jax.numpy as jnp
from jax import lax
from jax.experimental import pallas as pl
from jax.experimental.pallas import tpu as pltpu
from jax.experimental.pallas import tpu_sc as plsc

B, N, M, C = 8, 8192, 512, 64
S = 32
R2 = 0.1 * 0.1
NW = 32
GPW = (B * M) // NW
BLK = 8
NBLK = N // (16 * BLK)
LCAP = 208
TRASH = LCAP - 1
GQ = 8

_MESH = plsc.VectorSubcoreMesh(
    core_axis_name="c", subcore_axis_name="s", num_cores=2, num_subcores=16
)


@functools.partial(
    pl.kernel,
    out_type=(
        jax.ShapeDtypeStruct((B, C + 2, M, S), jnp.float32),
        jax.ShapeDtypeStruct((B * M * S,), jnp.int32),
    ),
    mesh=_MESH,
    compiler_params=pltpu.CompilerParams(needs_layout_passes=False),
    scratch_types=[
        pltpu.VMEM((N,), jnp.float32),
        pltpu.VMEM((N,), jnp.float32),
        pltpu.VMEM((GPW,), jnp.float32),
        pltpu.VMEM((GPW,), jnp.float32),
        pltpu.VMEM((GQ * LCAP,), jnp.int32),
        pltpu.VMEM((M * S,), jnp.int32),
        pltpu.VMEM((N,), jnp.float32),
        pltpu.VMEM((N,), jnp.float32),
        pltpu.VMEM((M,), jnp.float32),
        pltpu.VMEM((M // 2, S), jnp.float32),
        pltpu.VMEM((M // 2, S), jnp.float32),
        pltpu.SemaphoreType.DMA,
        pltpu.SemaphoreType.DMA,
        pltpu.SemaphoreType.DMA,
        pltpu.SemaphoreType.DMA,
    ],
)
def _qg_kernel(xs_h, ys_h, cx_h, cy_h, feat_h, out_h, idx_h,
               xs_v, ys_v, cx_v, cy_v, lst_v, idx_v, row0_v, row1_v, c_v,
               o0_v, o1_v, in_sem0, in_sem1, out_sem0, out_sem1):
    cid = lax.axis_index("c")
    sid = lax.axis_index("s")
    wid = cid * 16 + sid
    b = wid // 4
    lb = sid // 4
    q = wid % 4

    pltpu.sync_copy(xs_h.at[pl.ds(b * N, N)], xs_v)
    pltpu.sync_copy(ys_h.at[pl.ds(b * N, N)], ys_v)
    g0 = wid * GPW
    pltpu.sync_copy(cx_h.at[pl.ds(g0, GPW)], cx_v)
    pltpu.sync_copy(cy_h.at[pl.ds(g0, GPW)], cy_v)

    lane = lax.iota(jnp.int32, 16)
    r2 = jnp.float32(R2)

    def per_16_groups(gb, carry):
        cxv = cx_v[pl.ds(gb * 16, 16)]
        cyv = cy_v[pl.ds(gb * 16, 16)]
        for tq in range(16 // GQ):
            cxs = [cxv[tq * GQ + k] for k in range(GQ)]
            cys = [cyv[tq * GQ + k] for k in range(GQ)]

            def cond(state):
                blk = state[0]
                cnts = state[1:]
                mns = [c[0] for c in cnts]
                while len(mns) > 1:
                    mns = [jnp.minimum(a, b) for a, b in zip(mns[::2], mns[1::2])]
                return jnp.logical_and(mns[0] < S, blk < NBLK)

            def body(state):
                blk = state[0]
                cnts = list(state[1:])
                for tc in range(BLK):
                    base = (blk * BLK + tc) * 16
                    xv = xs_v[pl.ds(base, 16)]
                    yv = ys_v[pl.ds(base, 16)]
                    inds = lane + base
                    for k in range(GQ):
                        dx = xv - cxs[k]
                        dy = yv - cys[k]
                        msk = dx * dx + dy * dy < r2
                        pos = (cnts[k] + (k * LCAP - 1)) + plsc.cumsum(
                            msk.astype(jnp.int32)
                        )
                        pos = jnp.where(msk, pos, k * LCAP + TRASH)
                        plsc.store_scatter(lst_v, [pos], inds)
                        cnts[k] = cnts[k] + plsc.all_reduce_population_count(msk)
                return (blk + 1, *cnts)

            z = jnp.zeros((16,), jnp.int32)
            state = lax.while_loop(cond, body, (jnp.int32(0),) + (z,) * GQ)
            cnts = state[1:]
            for k in range(GQ):
                cnt_s = cnts[k][0]
                v0 = lst_v[pl.ds(k * LCAP, 16)]
                v1 = lst_v[pl.ds(k * LCAP + 16, 16)]
                f_s = jnp.where(cnt_s > 0, v0[0], 0)
                o0 = jnp.where(lane < cnt_s, v0, f_s)
                o1 = jnp.where(lane + 16 < cnt_s, v1, f_s)
                g = gb * 16 + tq * GQ + k
                idx_v[pl.ds(g * S, 16)] = o0
                idx_v[pl.ds(g * S + 16, 16)] = o1
        return carry

    lax.fori_loop(0, GPW // 16, per_16_groups, 0)

    pltpu.sync_copy(idx_v.at[pl.ds(0, GPW * S)], idx_h.at[pl.ds(wid * GPW * S, GPW * S)])
    plsc.subcore_barrier()
    pltpu.sync_copy(idx_h.at[pl.ds(b * M * S, M * S)], idx_v)

    HM = M // 2
    rows = [row0_v, row1_v]
    outs = [o0_v, o1_v]
    in_sems = [in_sem0, in_sem1]
    out_sems = [out_sem0, out_sem1]
    CH = C // 4

    def gather_half(row_v, o_v, mh):
        def gather_row(mm, _):
            m = mh * HM + mm
            for h in range(2):
                iv = idx_v[pl.ds(m * S + h * 16, 16)]
                o_v[mm, pl.ds(h * 16, 16)] = plsc.load_gather(row_v, [iv])
            return 0

        lax.fori_loop(0, HM, gather_row, 0, unroll=8)

    in_copies = [None, None]
    out_copies = [None, None]

    def start_in(ci):
        ch = q * CH + ci
        cp = pltpu.make_async_copy(
            feat_h.at[pl.ds((b * C + ch) * N, N)], rows[ci % 2], in_sems[ci % 2]
        )
        cp.start()
        in_copies[ci % 2] = cp

    start_in(0)
    for ci in range(CH):
        cur = ci % 2
        if ci + 1 < CH:
            start_in(ci + 1)
        in_copies[cur].wait()
        ch = q * CH + ci
        for mh in range(2):
            if out_copies[mh] is not None:
                out_copies[mh].wait()
            gather_half(rows[cur], outs[mh], mh)
            cp = pltpu.make_async_copy(
                outs[mh], out_h.at[b, 2 + ch, pl.ds(mh * HM, HM)], out_sems[mh]
            )
            cp.start()
            out_copies[mh] = cp
    out_copies[0].wait()
    out_copies[1].wait()

    def xyz_channel(src_v, cc_h, ch):
        pltpu.sync_copy(cc_h.at[pl.ds(b * M, M)], c_v)

        for mh in range(2):
            def body(mb, _):
                cv = c_v[pl.ds(mh * HM + mb * 16, 16)]
                for t in range(16):
                    c_s = cv[t]
                    mm = mb * 16 + t
                    base = (mh * HM + mm) * S
                    for h in range(2):
                        iv = idx_v[pl.ds(base + h * 16, 16)]
                        vals = plsc.load_gather(src_v, [iv]) - c_s
                        o0_v[mm, pl.ds(h * 16, 16)] = vals
                return 0

            lax.fori_loop(0, HM // 16, body, 0)
            pltpu.sync_copy(o0_v, out_h.at[b, ch, pl.ds(mh * HM, HM)])

    @pl.when(q == 0)
    def _():
        xyz_channel(xs_v, cx_h, 0)

    @pl.when(q == 1)
    def _():
        xyz_channel(ys_v, cy_h, 1)


def kernel(xyz, new_xyz, features):
    xs = xyz[:, :, 0].reshape(-1)
    ys = xyz[:, :, 1].reshape(-1)
    cx = new_xyz[:, :, 0].reshape(-1)
    cy = new_xyz[:, :, 1].reshape(-1)
    feat = features.reshape(-1)
    out, _ = _qg_kernel(xs, ys, cx, cy, feat)
    return out


# speedup vs baseline: 1.0104x; 1.0104x over previous
import functools

import jax
import jax.numpy as jnp
from jax import lax
from jax.experimental import pallas as pl
from jax.experimental.pallas import tpu as pltpu
from jax.experimental.pallas import tpu_sc as plsc

B, N, M, C = 8, 8192, 512, 64
S = 32
R2 = 0.1 * 0.1
NW = 32
GPW = (B * M) // NW
BLK = 8
NBLK = N // (16 * BLK)
LCAP = 208
TRASH = LCAP - 1
GQ = 8

_MESH = plsc.VectorSubcoreMesh(
    core_axis_name="c", subcore_axis_name="s", num_cores=2, num_subcores=16
)


@functools.partial(
    pl.kernel,
    out_type=(
        jax.ShapeDtypeStruct((B, C + 2, M, S), jnp.float32),
        jax.ShapeDtypeStruct((B * M * S,), jnp.int32),
    ),
    mesh=_MESH,
    compiler_params=pltpu.CompilerParams(needs_layout_passes=False),
    scratch_types=[
        pltpu.VMEM((N,), jnp.float32),
        pltpu.VMEM((N,), jnp.float32),
        pltpu.VMEM((GPW,), jnp.float32),
        pltpu.VMEM((GPW,), jnp.float32),
        pltpu.VMEM((GQ * LCAP,), jnp.int32),
        pltpu.VMEM((M * S,), jnp.int32),
        pltpu.VMEM((N,), jnp.float32),
        pltpu.VMEM((N,), jnp.float32),
        pltpu.VMEM((M,), jnp.float32),
        pltpu.VMEM((M // 2, S), jnp.float32),
        pltpu.VMEM((M // 2, S), jnp.float32),
        pltpu.SemaphoreType.DMA,
        pltpu.SemaphoreType.DMA,
        pltpu.SemaphoreType.DMA,
        pltpu.SemaphoreType.DMA,
    ],
)
def _qg_kernel(xs_h, ys_h, cx_h, cy_h, feat_h, out_h, idx_h,
               xs_v, ys_v, cx_v, cy_v, lst_v, idx_v, row0_v, row1_v, c_v,
               o0_v, o1_v, in_sem0, in_sem1, out_sem0, out_sem1):
    cid = lax.axis_index("c")
    sid = lax.axis_index("s")
    wid = cid * 16 + sid
    b = wid // 4
    lb = sid // 4
    q = wid % 4

    pltpu.sync_copy(xs_h.at[pl.ds(b * N, N)], xs_v)
    pltpu.sync_copy(ys_h.at[pl.ds(b * N, N)], ys_v)
    g0 = wid * GPW
    pltpu.sync_copy(cx_h.at[pl.ds(g0, GPW)], cx_v)
    pltpu.sync_copy(cy_h.at[pl.ds(g0, GPW)], cy_v)

    lane = lax.iota(jnp.int32, 16)
    r2 = jnp.float32(R2)

    def per_16_groups(gb, carry):
        cxv = cx_v[pl.ds(gb * 16, 16)]
        cyv = cy_v[pl.ds(gb * 16, 16)]
        for tq in range(16 // GQ):
            cxs = [cxv[tq * GQ + k] for k in range(GQ)]
            cys = [cyv[tq * GQ + k] for k in range(GQ)]

            def cond(state):
                blk = state[0]
                cnts = state[1:]
                mns = [c[0] for c in cnts]
                while len(mns) > 1:
                    mns = [jnp.minimum(a, b) for a, b in zip(mns[::2], mns[1::2])]
                return jnp.logical_and(mns[0] < S, blk < NBLK)

            def body(state):
                blk = state[0]
                cnts = list(state[1:])
                for tc in range(BLK):
                    base = (blk * BLK + tc) * 16
                    xv = xs_v[pl.ds(base, 16)]
                    yv = ys_v[pl.ds(base, 16)]
                    inds = lane + base
                    for k in range(GQ):
                        dx = xv - cxs[k]
                        dy = yv - cys[k]
                        msk = dx * dx + dy * dy < r2
                        pos = (cnts[k] + (k * LCAP - 1)) + plsc.cumsum(
                            msk.astype(jnp.int32)
                        )
                        pos = jnp.where(msk, pos, k * LCAP + TRASH)
                        plsc.store_scatter(lst_v, [pos], inds)
                        cnts[k] = cnts[k] + plsc.all_reduce_population_count(msk)
                return (blk + 1, *cnts)

            z = jnp.zeros((16,), jnp.int32)
            state = lax.while_loop(cond, body, (jnp.int32(0),) + (z,) * GQ)
            cnts = state[1:]
            for k in range(GQ):
                cnt_s = cnts[k][0]
                v0 = lst_v[pl.ds(k * LCAP, 16)]
                v1 = lst_v[pl.ds(k * LCAP + 16, 16)]
                f_s = jnp.where(cnt_s > 0, v0[0], 0)
                o0 = jnp.where(lane < cnt_s, v0, f_s)
                o1 = jnp.where(lane + 16 < cnt_s, v1, f_s)
                g = gb * 16 + tq * GQ + k
                idx_v[pl.ds(g * S, 16)] = o0
                idx_v[pl.ds(g * S + 16, 16)] = o1
        return carry

    lax.fori_loop(0, GPW // 16, per_16_groups, 0)

    pltpu.sync_copy(idx_v.at[pl.ds(0, GPW * S)], idx_h.at[pl.ds(wid * GPW * S, GPW * S)])
    plsc.subcore_barrier()
    pltpu.sync_copy(idx_h.at[pl.ds(b * M * S, M * S)], idx_v)

    HM = M // 2
    rows = [row0_v, row1_v]
    outs = [o0_v, o1_v]
    in_sems = [in_sem0, in_sem1]
    out_sems = [out_sem0, out_sem1]
    CH = C // 4

    def gather_pair_half(mh):
        def gather_row(mm, _):
            m = mh * HM + mm
            for h in range(2):
                iv = idx_v[pl.ds(m * S + h * 16, 16)]
                o0_v[mm, pl.ds(h * 16, 16)] = plsc.load_gather(row0_v, [iv])
                o1_v[mm, pl.ds(h * 16, 16)] = plsc.load_gather(row1_v, [iv])
            return 0

        lax.fori_loop(0, HM, gather_row, 0, unroll=4)

    out_copies = [None, None]
    for pi in range(CH // 2):
        chA = q * CH + 2 * pi
        inA = pltpu.make_async_copy(
            feat_h.at[pl.ds((b * C + chA) * N, N)], row0_v, in_sem0
        )
        inB = pltpu.make_async_copy(
            feat_h.at[pl.ds((b * C + chA + 1) * N, N)], row1_v, in_sem1
        )
        inA.start()
        inB.start()
        inA.wait()
        inB.wait()
        for mh in range(2):
            for oc in out_copies:
                if oc is not None:
                    oc.wait()
            gather_pair_half(mh)
            cpA = pltpu.make_async_copy(
                o0_v, out_h.at[b, 2 + chA, pl.ds(mh * HM, HM)], out_sem0
            )
            cpB = pltpu.make_async_copy(
                o1_v, out_h.at[b, 3 + chA, pl.ds(mh * HM, HM)], out_sem1
            )
            cpA.start()
            cpB.start()
            out_copies = [cpA, cpB]
    for oc in out_copies:
        if oc is not None:
            oc.wait()

    def xyz_channel(src_v, cc_h, ch):
        pltpu.sync_copy(cc_h.at[pl.ds(b * M, M)], c_v)

        for mh in range(2):
            def body(mb, _):
                cv = c_v[pl.ds(mh * HM + mb * 16, 16)]
                for t in range(16):
                    c_s = cv[t]
                    mm = mb * 16 + t
                    base = (mh * HM + mm) * S
                    for h in range(2):
                        iv = idx_v[pl.ds(base + h * 16, 16)]
                        vals = plsc.load_gather(src_v, [iv]) - c_s
                        o0_v[mm, pl.ds(h * 16, 16)] = vals
                return 0

            lax.fori_loop(0, HM // 16, body, 0)
            pltpu.sync_copy(o0_v, out_h.at[b, ch, pl.ds(mh * HM, HM)])

    @pl.when(q == 0)
    def _():
        xyz_channel(xs_v, cx_h, 0)

    @pl.when(q == 1)
    def _():
        xyz_channel(ys_v, cy_h, 1)


def kernel(xyz, new_xyz, features):
    xs = xyz[:, :, 0].reshape(-1)
    ys = xyz[:, :, 1].reshape(-1)
    cx = new_xyz[:, :, 0].reshape(-1)
    cy = new_xyz[:, :, 1].reshape(-1)
    feat = features.reshape(-1)
    out, _ = _qg_kernel(xs, ys, cx, cy, feat)
    return out
